# Initial kernel scaffold; baseline (speedup 1.0000x reference)
import jax
import jax.numpy as jnp
from jax.experimental import pallas as pl


def kernel(hidden, edge_index, edge_weights, ln_gamma, ln_beta, W_gat, att_src, att_dst, W_gcn, b_gcn):
    raise NotImplementedError("write your pallas kernel here")


# speedup vs baseline: 58.1435x; 58.1435x over previous
import functools

import jax
import jax.numpy as jnp
from jax import lax
from jax.experimental import pallas as pl
from jax.experimental.pallas import tpu as pltpu
from jax.experimental.pallas import tpu_sc as plsc

N = 10000
E = 320000
D = 128
H = 4

NP = 10240
TW = 144
OW = 160
AW = 16
CH = 112
NWORK = 32
NCH = 90
EPW = NCH * CH
EP = EPW * NWORK
ZSH = NP // 16
ZCH = 80
BLK = 1024
GRID = NP // BLK

_f32 = jnp.float32
_i32 = jnp.int32

_GDN = lax.GatherDimensionNumbers(
    offset_dims=(), collapsed_slice_dims=(0,), start_index_map=(0,))


def _dyn_gather(vec, idx):
    return lax.gather(vec, idx[:, None], _GDN, slice_sizes=(1,),
                      mode=lax.GatherScatterMode.PROMISE_IN_BOUNDS)


def _tca_body(h_ref, gam_ref, bet_ref, w2_ref, o_ref):
    x = h_ref[...]
    mu = jnp.mean(x, axis=-1, keepdims=True)
    xc = x - mu
    var = jnp.mean(xc * xc, axis=-1, keepdims=True)
    hn = xc * lax.rsqrt(var + 1e-12) * gam_ref[...] + bet_ref[...]
    o_ref[...] = jnp.dot(hn, w2_ref[...], preferred_element_type=_f32)


def _run_tca(hp, gam, bet, w2):
    return pl.pallas_call(
        _tca_body,
        grid=(GRID,),
        in_specs=[
            pl.BlockSpec((BLK, D), lambda i: (i, 0)),
            pl.BlockSpec((1, D), lambda i: (0, 0)),
            pl.BlockSpec((1, D), lambda i: (0, 0)),
            pl.BlockSpec((D, OW), lambda i: (0, 0)),
        ],
        out_specs=pl.BlockSpec((BLK, OW), lambda i: (i, 0)),
        out_shape=jax.ShapeDtypeStruct((NP, OW), _f32),
    )(hp, gam, bet, w2)


def _tcb_body(accp_ref, o_ref, c_ref, wg_ref, t2_ref, dinv_ref):
    acc = accp_ref[0] + accp_ref[1]
    o = o_ref[...]
    s = o[:, :D]
    a = o[:, D:D + H] + o[:, D + H:D + 2 * H]
    al = jnp.where(a >= 0, a, 0.2 * a)
    el = jnp.exp(al - c_ref[:, :H])
    asum = acc[:, D:D + H] + el
    hh = lax.broadcasted_iota(_i32, (H, D), 0)
    dd = lax.broadcasted_iota(_i32, (H, D), 1) // 32
    rep = (hh == dd).astype(_f32)
    elb = jnp.dot(el, rep, preferred_element_type=_f32)
    asb = jnp.dot(asum, rep, preferred_element_type=_f32)
    g = (acc[:, :D] + elb * s) / (asb + 1e-16)
    deg = acc[:, D + H:D + H + 1]
    dinv = jnp.where(deg > 0, lax.rsqrt(deg), 0.0)
    xw2 = jnp.dot(g, wg_ref[...], preferred_element_type=_f32)
    t2_ref[...] = xw2 * dinv
    dinv_ref[...] = jnp.broadcast_to(dinv, (BLK, 8))


def _run_tcb(accp, o, cvec, wg):
    return pl.pallas_call(
        _tcb_body,
        grid=(GRID,),
        in_specs=[
            pl.BlockSpec((2, BLK, TW), lambda i: (0, i, 0)),
            pl.BlockSpec((BLK, OW), lambda i: (i, 0)),
            pl.BlockSpec((1, D), lambda i: (0, 0)),
            pl.BlockSpec((D, D), lambda i: (0, 0)),
        ],
        out_specs=[
            pl.BlockSpec((BLK, D), lambda i: (i, 0)),
            pl.BlockSpec((BLK, 8), lambda i: (i, 0)),
        ],
        out_shape=[
            jax.ShapeDtypeStruct((NP, D), _f32),
            jax.ShapeDtypeStruct((NP, 8), _f32),
        ],
    )(accp, o, cvec, wg)


def _tcc_body(acc2_ref, dinv_ref, b_ref, out_ref):
    acc = acc2_ref[0] + acc2_ref[1]
    out_ref[...] = acc * dinv_ref[:, 0:1] + b_ref[...]


def _run_tcc(acc2, dinvb, b):
    return pl.pallas_call(
        _tcc_body,
        grid=(GRID,),
        in_specs=[
            pl.BlockSpec((2, BLK, D), lambda i: (0, i, 0)),
            pl.BlockSpec((BLK, 8), lambda i: (i, 0)),
            pl.BlockSpec((1, D), lambda i: (0, 0)),
        ],
        out_specs=pl.BlockSpec((BLK, D), lambda i: (i, 0)),
        out_shape=jax.ShapeDtypeStruct((NP, D), _f32),
    )(acc2, dinvb, b)


def _sc1_body(t1, a2, cv_in, epk3, out,
              ebuf, rows, adrows, cvv, acc, sem1, sem2):
    cid = lax.axis_index("c")
    sid = lax.axis_index("s")
    wid = cid * 16 + sid

    pltpu.sync_copy(cv_in, cvv)
    cvec = cvv[...]
    lanes = lax.iota(_i32, 16)
    hsel = [jnp.full((16,), h, dtype=_i32) for h in range(H)]
    jsel = [jnp.full((16,), j, dtype=_i32) for j in range(16)]
    zv = jnp.zeros((16,), dtype=_f32)

    def _zrow(r, c):
        for cg in range(TW // 16):
            rows[0, r, pl.ds(cg * 16, 16)] = zv
        return c
    lax.fori_loop(0, ZCH, _zrow, 0)
    for k in range(ZSH // ZCH):
        pltpu.sync_copy(rows.at[0].at[pl.ds(0, ZCH)],
                        acc.at[pl.ds(sid * ZSH + k * ZCH, ZCH)])
    plsc.subcore_barrier()

    def _issue(i, b):
        pltpu.sync_copy(epk3.at[wid * NCH + i], ebuf.at[b])
        pltpu.async_copy(t1.at[ebuf.at[b, 0]], rows.at[b], sem1)
        pltpu.async_copy(a2.at[ebuf.at[b, 1]], adrows.at[b], sem2)

    _issue(0, 0)

    def _pair(i2, c):
        for b in range(2):
            i = i2 * 2 + b
            pltpu.make_async_copy(t1.at[ebuf.at[b, 0]], rows.at[b],
                                  sem1).wait()
            pltpu.make_async_copy(a2.at[ebuf.at[b, 1]], adrows.at[b],
                                  sem2).wait()

            @pl.when(i + 1 < NCH)
            def _():
                _issue(i + 1, 1 - b)

            for g in range(CH // 16):
                w16 = lax.bitcast_convert_type(
                    ebuf[b, 2, pl.ds(g * 16, 16)], _f32)
                for j in range(16):
                    e = g * 16 + j
                    sv = rows[b, e, pl.ds(D, 16)]
                    dv = adrows[b, e, pl.ds(0, 16)]
                    a = sv + dv
                    al = jnp.where(a >= 0, a, 0.2 * a)
                    eal = jnp.exp(al - cvec)
                    wb = _dyn_gather(w16, jsel[j])
                    tail = jnp.where(lanes < 4, eal,
                                     jnp.where(lanes == 4, wb, zv))
                    rows[b, e, pl.ds(D, 16)] = tail
                    ms = [_dyn_gather(eal, hsel[h]) for h in range(H)]
                    for r in range(D // 16):
                        rows[b, e, pl.ds(r * 16, 16)] = (
                            rows[b, e, pl.ds(r * 16, 16)] * ms[r // 2])
            pltpu.sync_copy(rows.at[b], acc.at[ebuf.at[b, 1]], add=True)
        return c
    lax.fori_loop(0, NCH // 2, _pair, 0)

    plsc.subcore_barrier()
    for k in range(ZSH // ZCH):
        r0 = sid * ZSH + k * ZCH
        pltpu.sync_copy(acc.at[pl.ds(r0, ZCH)], out.at[cid, pl.ds(r0, ZCH)])


def _run_sc1(t1, a2, cvec, epk3):
    mesh = plsc.VectorSubcoreMesh(core_axis_name="c", subcore_axis_name="s")
    fn = functools.partial(
        pl.kernel,
        out_type=jax.ShapeDtypeStruct((2, NP, TW), _f32),
        mesh=mesh,
        compiler_params=pltpu.CompilerParams(use_tc_tiling_on_sc=False),
        scratch_types=[
            pltpu.VMEM((2, 3, CH), _i32),
            pltpu.VMEM((2, CH, TW), _f32),
            pltpu.VMEM((2, CH, AW), _f32),
            pltpu.VMEM((16,), _f32),
            pltpu.VMEM_SHARED((NP, TW), _f32),
            pltpu.SemaphoreType.DMA,
            pltpu.SemaphoreType.DMA,
        ],
    )(_sc1_body)
    return fn(t1, a2, cvec, epk3)


def _sc2_body(t2, epk3, out,
              ebuf, rows, acc, sem1):
    cid = lax.axis_index("c")
    sid = lax.axis_index("s")
    wid = cid * 16 + sid
    zv = jnp.zeros((16,), dtype=_f32)
    jsel = [jnp.full((16,), j, dtype=_i32) for j in range(16)]

    def _zrow(r, c):
        for cg in range(D // 16):
            rows[0, r, pl.ds(cg * 16, 16)] = zv
        return c
    lax.fori_loop(0, ZCH, _zrow, 0)
    for k in range(ZSH // ZCH):
        pltpu.sync_copy(rows.at[0].at[pl.ds(0, ZCH)],
                        acc.at[pl.ds(sid * ZSH + k * ZCH, ZCH)])
    plsc.subcore_barrier()

    def _issue(i, b):
        pltpu.sync_copy(epk3.at[wid * NCH + i], ebuf.at[b])
        pltpu.async_copy(t2.at[ebuf.at[b, 0]], rows.at[b], sem1)

    _issue(0, 0)

    def _pair(i2, c):
        for b in range(2):
            i = i2 * 2 + b
            pltpu.make_async_copy(t2.at[ebuf.at[b, 0]], rows.at[b],
                                  sem1).wait()

            @pl.when(i + 1 < NCH)
            def _():
                _issue(i + 1, 1 - b)

            for g in range(CH // 16):
                w16 = lax.bitcast_convert_type(
                    ebuf[b, 2, pl.ds(g * 16, 16)], _f32)
                for j in range(16):
                    e = g * 16 + j
                    wb = _dyn_gather(w16, jsel[j])
                    for r in range(D // 16):
                        rows[b, e, pl.ds(r * 16, 16)] = (
                            rows[b, e, pl.ds(r * 16, 16)] * wb)
            pltpu.sync_copy(rows.at[b], acc.at[ebuf.at[b, 1]], add=True)
        return c
    lax.fori_loop(0, NCH // 2, _pair, 0)

    plsc.subcore_barrier()
    for k in range(ZSH // ZCH):
        r0 = sid * ZSH + k * ZCH
        pltpu.sync_copy(acc.at[pl.ds(r0, ZCH)], out.at[cid, pl.ds(r0, ZCH)])


def _run_sc2(t2, epk3):
    mesh = plsc.VectorSubcoreMesh(core_axis_name="c", subcore_axis_name="s")
    fn = functools.partial(
        pl.kernel,
        out_type=jax.ShapeDtypeStruct((2, NP, D), _f32),
        mesh=mesh,
        compiler_params=pltpu.CompilerParams(use_tc_tiling_on_sc=False),
        scratch_types=[
            pltpu.VMEM((2, 3, CH), _i32),
            pltpu.VMEM((2, CH, D), _f32),
            pltpu.VMEM_SHARED((NP, D), _f32),
            pltpu.SemaphoreType.DMA,
        ],
    )(_sc2_body)
    return fn(t2, epk3)


def kernel(hidden, edge_index, edge_weights, ln_gamma, ln_beta, W_gat,
           att_src, att_dst, W_gcn, b_gcn):
    wg4 = W_gat.reshape(D, H, D // H, H).sum(axis=-1) * (1.0 / H)
    wgr = W_gat.reshape(D, H, D)
    a_s = jnp.einsum("dhe,he->dh", wgr, att_src)
    a_d = jnp.einsum("dhe,he->dh", wgr, att_dst)
    w2 = jnp.concatenate(
        [wg4.reshape(D, D), a_s, a_d,
         jnp.zeros((D, OW - D - 2 * H), _f32)], axis=1)

    hp = jnp.concatenate([hidden, jnp.zeros((NP - N, D), _f32)], axis=0)
    o = _run_tca(hp, ln_gamma.reshape(1, D), ln_beta.reshape(1, D), w2)
    o = o.at[N:].set(0.0)

    cmax = jnp.max(o[:, D:D + 2 * H], axis=0)
    csum = cmax[:H] + cmax[H:]
    c4 = jnp.where(csum >= 0, csum, 0.2 * csum)
    cvec16 = jnp.concatenate([c4, jnp.zeros((12,), _f32)])
    cvec = jnp.concatenate([c4, jnp.zeros((D - H,), _f32)]).reshape(1, D)

    t1 = o[:, :TW]
    a2 = o[:, D + H:D + H + AW]

    pad = EP - E
    srcp = jnp.concatenate([edge_index[0], jnp.full((pad,), N, _i32)])
    dstp = jnp.concatenate([edge_index[1], jnp.full((pad,), N, _i32)])
    wp = jnp.concatenate([edge_weights, jnp.zeros((pad,), _f32)])
    epk3 = jnp.stack([srcp, dstp, lax.bitcast_convert_type(wp, _i32)]
                     ).reshape(3, NWORK * NCH, CH).transpose(1, 0, 2)

    accp = _run_sc1(t1, a2, cvec16, epk3)
    t2, dinvb = _run_tcb(accp, o, cvec, W_gcn)
    acc2 = _run_sc2(t2, epk3)
    outp = _run_tcc(acc2, dinvb, b_gcn.reshape(1, D))
    return (outp[:N], edge_index, edge_weights)
